# baseline (speedup 1.0000x reference)
import functools
import jax
import jax.numpy as jnp
from jax import lax
from jax.experimental import pallas as pl
from jax.experimental.pallas import tpu as pltpu
from jax.experimental.pallas import tpu_sc as plsc

NC = 2
NS = 16
NW = NC * NS

VOCAB_ROWS = 1_000_000
DIM = 64
BATCH = 4096 * 200
B_PER_W = BATCH // NW
CHUNK = 512
K = 1
SUPER = CHUNK * K
N_SUPER = B_PER_W // SUPER
NBUF = 2


def _emb_body(x_hbm, table_hbm, out_hbm, idx_v, rows_v, gsems, osems):
    wid = lax.axis_index("s") * NC + lax.axis_index("c")
    base = pl.multiple_of(wid * B_PER_W, B_PER_W)

    pltpu.sync_copy(x_hbm.at[pl.ds(base, B_PER_W)], idx_v)

    def gather(g, buf, wait):
        for j in range(K):
            cp = pltpu.make_async_copy(
                table_hbm.at[idx_v.at[pl.ds(
                    pl.multiple_of((g * K + j) * CHUNK, CHUNK), CHUNK)]],
                rows_v.at[buf, pl.ds(j * CHUNK, CHUNK)],
                gsems.at[buf],
            )
            cp.wait() if wait else cp.start()

    def out_copy(g, buf, wait):
        cp = pltpu.make_async_copy(
            rows_v.at[buf],
            out_hbm.at[(base // SUPER) + g],
            osems.at[buf],
        )
        cp.wait() if wait else cp.start()

    for b in range(NBUF):
        gather(b, b, False)

    def outer(i, carry):
        g0 = i * NBUF
        for b in range(NBUF):
            g = g0 + b
            gather(g, b, True)
            out_copy(g, b, False)
            out_copy(g, b, True)

            @pl.when(g + NBUF < N_SUPER)
            def _():
                gather(g + NBUF, b, False)

        return carry

    lax.fori_loop(0, N_SUPER // NBUF, outer, 0, unroll=False)


@jax.jit
def _emb(x_flat, table):
    run = pl.kernel(
        _emb_body,
        out_type=jax.ShapeDtypeStruct((BATCH // SUPER, SUPER, DIM),
                                      jnp.float32),
        mesh=plsc.VectorSubcoreMesh(core_axis_name="c", subcore_axis_name="s"),
        scratch_types=[
            pltpu.VMEM((B_PER_W,), jnp.int32),
            pltpu.VMEM((NBUF, SUPER, DIM), jnp.float32),
            pltpu.SemaphoreType.DMA((NBUF,)),
            pltpu.SemaphoreType.DMA((NBUF,)),
        ],
        compiler_params=pltpu.CompilerParams(use_tc_tiling_on_sc=False),
    )
    return run(x_flat, table)


def kernel(x, table):
    out = _emb(x.reshape(-1), table)
    return out.reshape(x.shape[0], x.shape[1], DIM)


# speedup vs baseline: 1.3295x; 1.3295x over previous
import jax
import jax.numpy as jnp
from jax import lax
from jax.experimental import pallas as pl
from jax.experimental.pallas import tpu as pltpu
from jax.experimental.pallas import tpu_sc as plsc

NC = 2
NS = 16
NW = NC * NS

VOCAB_ROWS = 1_000_000
DIM = 64
BATCH = 4096 * 200
B_PER_W = BATCH // NW
CHUNK = 200
RPB = 2
SUPER = CHUNK * RPB
N_SUPER = B_PER_W // SUPER
NBUF = 2


def _emb_body(x_hbm, table_hbm, out_hbm, idx_v, rows_v, gsems, osems):
    wid = lax.axis_index("s") * NC + lax.axis_index("c")
    base = pl.multiple_of(wid * B_PER_W, B_PER_W)

    pltpu.sync_copy(x_hbm.at[pl.ds(base, B_PER_W)], idx_v)

    def gather(g, buf, wait):
        for j in range(RPB):
            cp = pltpu.make_async_copy(
                table_hbm.at[idx_v.at[pl.ds(
                    pl.multiple_of((g * RPB + j) * CHUNK, CHUNK), CHUNK)]],
                rows_v.at[buf, j],
                gsems.at[buf],
            )
            cp.wait() if wait else cp.start()

    def out_copy(g, buf, wait):
        cp = pltpu.make_async_copy(
            rows_v.at[buf],
            out_hbm.at[pl.ds(pl.multiple_of((base // CHUNK) + g * RPB, RPB),
                             RPB), :, pl.ds(0, DIM)],
            osems.at[buf],
        )
        cp.wait() if wait else cp.start()

    for b in range(NBUF):
        gather(b, b, False)

    def outer(i, carry):
        g0 = i * NBUF
        for b in range(NBUF):
            g = g0 + b
            gather(g, b, True)
            out_copy(g, b, False)
            out_copy(g, b, True)

            @pl.when(g + NBUF < N_SUPER)
            def _():
                gather(g + NBUF, b, False)

        return carry

    lax.fori_loop(0, N_SUPER // NBUF, outer, 0, unroll=False)


@jax.jit
def _emb(x_flat, table):
    run = pl.kernel(
        _emb_body,
        out_type=jax.ShapeDtypeStruct((4096, CHUNK, 2 * DIM), jnp.float32),
        mesh=plsc.VectorSubcoreMesh(core_axis_name="c", subcore_axis_name="s"),
        scratch_types=[
            pltpu.VMEM((B_PER_W,), jnp.int32),
            pltpu.VMEM((NBUF, RPB, CHUNK, DIM), jnp.float32),
            pltpu.SemaphoreType.DMA((NBUF,)),
            pltpu.SemaphoreType.DMA((NBUF,)),
        ],
        compiler_params=pltpu.CompilerParams(use_tc_tiling_on_sc=False),
    )
    return run(x_flat, table)


def kernel(x, table):
    return _emb(x.reshape(-1), table)[:, :, :DIM]
